# baseline (speedup 1.0000x reference)
import functools

import jax
import jax.numpy as jnp
from jax import lax
from jax.experimental import pallas as pl
from jax.experimental.pallas import tpu as pltpu
from jax.experimental.pallas import tpu_sc as plsc

_N = 50000
_E = 800000
_G = 4096
_NP = 51200
_TPS = _NP // 16
_CW = 32
_XR = _N // 16
_NBLK = _E // 128
_NMAC = _NBLK // 5
_NPAIR = _NMAC // 2
_NPR3 = _NBLK // 6
_RB = 1000

_mesh = plsc.VectorSubcoreMesh(core_axis_name="c", subcore_axis_name="s")


@functools.partial(
    pl.kernel,
    out_type=jax.ShapeDtypeStruct((2 * _NP,), jnp.float32),
    mesh=_mesh,
    compiler_params=pltpu.CompilerParams(use_tc_tiling_on_sc=False),
    scratch_types=[
        pltpu.VMEM((5, 128), jnp.int32),
        pltpu.VMEM((5, 128), jnp.int32),
        pltpu.VMEM((128,), jnp.float32),
        pltpu.VMEM_SHARED((_NP,), jnp.float32),
        pltpu.SemaphoreType.DMA,
        pltpu.SemaphoreType.DMA,
    ],
)
def _deg_kernel(cols2_hbm, ones_hbm, z1_hbm, deg_out,
                ci0, ci1, ones_v, acc, isem, ssem):
    c = lax.axis_index("c")
    s = lax.axis_index("s")
    w = c * 16 + s
    pltpu.sync_copy(z1_hbm, acc.at[pl.ds(s * _TPS, _TPS)])
    pltpu.sync_copy(ones_hbm, ones_v)
    plsc.subcore_barrier()
    plo = (w * _NPAIR) // 32
    phi = ((w + 1) * _NPAIR) // 32

    def scat5(ci):
        sd = [pltpu.async_copy(ones_v, acc.at[ci.at[j]], ssem, add=True)
              for j in range(5)]
        for d in sd:
            d.wait()

    pltpu.sync_copy(cols2_hbm.at[pl.ds(2 * plo * 5, 5)], ci0)

    def pair(mp, _):
        m1 = 2 * mp + 1
        d1 = pltpu.async_copy(cols2_hbm.at[pl.ds(m1 * 5, 5)], ci1, isem)
        scat5(ci0)
        d1.wait()
        nxt = jnp.minimum(2 * mp + 2, _NMAC - 1)
        d0 = pltpu.async_copy(cols2_hbm.at[pl.ds(nxt * 5, 5)], ci0, isem)
        scat5(ci1)
        d0.wait()
        return 0

    lax.fori_loop(plo, phi, pair, 0)
    plsc.subcore_barrier()
    pltpu.sync_copy(acc.at[pl.ds(s * _TPS, _TPS)],
                    deg_out.at[pl.ds(c * _NP + s * _TPS, _TPS)])


def _make_agg(nchunks):
    per_core = nchunks // 2

    @functools.partial(
        pl.kernel,
        out_type=jax.ShapeDtypeStruct((_NP, 128), jnp.float32),
        mesh=_mesh,
        compiler_params=pltpu.CompilerParams(use_tc_tiling_on_sc=False),
        scratch_types=[
            pltpu.VMEM((3, 128), jnp.int32),
            pltpu.VMEM((3, 128), jnp.int32),
            pltpu.VMEM((3, 128), jnp.int32),
            pltpu.VMEM((3, 128), jnp.int32),
            pltpu.VMEM((3, 128, _CW), jnp.float32),
            pltpu.VMEM((3, 128, _CW), jnp.float32),
            pltpu.VMEM_SHARED((_NP, _CW), jnp.float32),
            pltpu.SemaphoreType.DMA,
            pltpu.SemaphoreType.DMA,
            pltpu.SemaphoreType.DMA,
            pltpu.SemaphoreType.DMA,
            pltpu.SemaphoreType.DMA,
        ],
    )
    def _agg(*args):
        rows4 = args[:nchunks]
        (cols2_hbm, z2_hbm, gflat_hbm, aggfull,
         riA, ciA, riB, ciB, bufA, bufB, acc,
         isem, gsemA, gsemB, ssemA, ssemB) = args[nchunks:]
        c = lax.axis_index("c")
        s = lax.axis_index("s")
        plo = (s * _NPR3) // 16
        phi = ((s + 1) * _NPR3) // 16

        def one_pass(chunk):
            col0 = _CW * chunk
            r4 = rows4[chunk]
            pltpu.sync_copy(z2_hbm, acc.at[pl.ds(s * _TPS, _TPS)])
            plsc.subcore_barrier()

            pltpu.sync_copy(r4.at[pl.ds(plo * 6, 3)], riA)
            pltpu.sync_copy(cols2_hbm.at[pl.ds(plo * 6, 3)], ciA)

            def pair(mp, _):
                b0 = mp * 6
                gdA = [pltpu.async_copy(gflat_hbm.at[riA.at[j]], bufA.at[j],
                                        gsemA) for j in range(3)]
                dBr = pltpu.async_copy(r4.at[pl.ds(b0 + 3, 3)], riB, isem)
                dBc = pltpu.async_copy(cols2_hbm.at[pl.ds(b0 + 3, 3)], ciB,
                                       isem)
                sdA = []
                for j in range(3):
                    gdA[j].wait()
                    sdA.append(pltpu.async_copy(bufA.at[j],
                                                acc.at[ciA.at[j]],
                                                ssemA, add=True))
                dBr.wait()
                dBc.wait()
                gdB = [pltpu.async_copy(gflat_hbm.at[riB.at[j]], bufB.at[j],
                                        gsemB) for j in range(3)]
                sdB = []
                for j in range(3):
                    gdB[j].wait()
                    sdB.append(pltpu.async_copy(bufB.at[j],
                                                acc.at[ciB.at[j]],
                                                ssemB, add=True))
                for d in sdA:
                    d.wait()
                nb = jnp.minimum((mp + 1) * 6, (_NPR3 - 1) * 6)
                dAr = pltpu.async_copy(r4.at[pl.ds(nb, 3)], riA, isem)
                dAc = pltpu.async_copy(cols2_hbm.at[pl.ds(nb, 3)], ciA, isem)
                for d in sdB:
                    d.wait()
                dAr.wait()
                dAc.wait()
                return 0

            lax.fori_loop(plo, phi, pair, 0)

            @pl.when(s == 15)
            def _():
                for t in range(_NBLK - 6 * _NPR3):
                    b = 6 * _NPR3 + t
                    pltpu.sync_copy(r4.at[pl.ds(b, 1)], riA.at[pl.ds(0, 1)])
                    pltpu.sync_copy(cols2_hbm.at[pl.ds(b, 1)],
                                    ciA.at[pl.ds(0, 1)])
                    pltpu.async_copy(gflat_hbm.at[riA.at[0]], bufA.at[0],
                                     gsemA).wait()
                    pltpu.sync_copy(bufA.at[0], acc.at[ciA.at[0]], add=True)

            plsc.subcore_barrier()
            pltpu.sync_copy(
                acc.at[pl.ds(s * _TPS, _TPS)],
                aggfull.at[pl.ds(s * _TPS, _TPS), pl.ds(col0, _CW)])

        for core_id in range(2):
            @pl.when(c == core_id)
            def _(core_id=core_id):
                for k in range(per_core):
                    one_pass(core_id * per_core + k)

    return _agg


_agg4 = _make_agg(4)
_agg2 = _make_agg(2)


@functools.partial(
    pl.kernel,
    out_type=jax.ShapeDtypeStruct((_G * 8,), jnp.float32),
    mesh=_mesh,
    compiler_params=pltpu.CompilerParams(use_tc_tiling_on_sc=False,
                                         needs_layout_passes=False),
    scratch_types=[
        pltpu.VMEM((_N + 16,), jnp.int32),
        pltpu.VMEM((16656,), jnp.float32),
        pltpu.VMEM((1040,), jnp.float32),
    ],
)
def _seg_kernel(h3f_hbm, batch_hbm, pf_out, batch_v, buf_v, outl_v):
    w = lax.axis_index("s") * 2 + lax.axis_index("c")
    g0 = w * 128
    pltpu.sync_copy(batch_hbm, batch_v.at[pl.ds(0, _N)])

    def _bat(i):
        return batch_v[pl.ds(i, 16)][0]

    def _lower_bound(target):
        def body(_, st):
            lo, hi = st
            mid = (lo + hi) // 2
            big = _bat(mid) >= target
            nlo = jnp.where(big, lo, mid + 1)
            nhi = jnp.where(big, mid, hi)
            keep = lo < hi
            return (jnp.where(keep, nlo, lo), jnp.where(keep, nhi, hi))

        return lax.fori_loop(0, 17, body, (jnp.int32(0), jnp.int32(_N)))[0]

    lo = _lower_bound(g0)
    hi = _lower_bound(g0 + 128)

    ones16 = jnp.ones((16,), jnp.float32)

    def initb(k, _):
        outl_v[pl.ds(k * 16, 16)] = ones16
        return 0

    lax.fori_loop(0, 65, initb, 0)

    lane = lax.iota(jnp.int32, 16)
    lmask = lane < 8
    nblk = (hi - lo + 127) // 128

    def outer(t, _):
        i0 = lo + t * 128
        pltpu.sync_copy(h3f_hbm.at[pl.ds(i0 * 128, 16640)], buf_v.at[pl.ds(0, 16640)])
        nn = jnp.minimum(hi - i0, 128)

        def inner(j, _2):
            b = _bat(i0 + j)
            idxv = (b - g0) * 8 + lane
            old = plsc.load_gather(outl_v, [idxv], mask=lmask)
            v = buf_v[pl.ds(j * 128, 16)]
            v = jnp.where(lmask, v, 1.0)
            old = jnp.where(lmask, old, 1.0)
            plsc.store_scatter(outl_v, [idxv], old * v, mask=lmask)
            return 0

        lax.fori_loop(0, nn, inner, 0)
        return 0

    lax.fori_loop(0, nblk, outer, 0)
    pltpu.sync_copy(outl_v.at[pl.ds(0, 1024)], pf_out.at[pl.ds(w * 1024, 1024)])


def _tc_a_body(x_ref, degp_ref, W1_ref, b1_ref, Wc1_ref, gf_ref, dis_ref):
    degp = degp_ref[...]
    deg = degp[:, 0] + degp[:, 1] + 1.0
    dis = lax.rsqrt(deg)[:, None]
    h = jnp.maximum(
        jnp.dot(x_ref[...], W1_ref[...], preferred_element_type=jnp.float32)
        + b1_ref[...], 0.0)
    t1 = jnp.dot(h, Wc1_ref[...], preferred_element_type=jnp.float32)
    g = dis * t1
    gf_ref[...] = jnp.concatenate(
        [g, jnp.zeros((_RB, 28), jnp.float32)], axis=1)
    dis_ref[...] = dis


def _tc_a(x, degpair, W1, b1, Wc1):
    return pl.pallas_call(
        _tc_a_body,
        grid=(_N // _RB,),
        in_specs=[
            pl.BlockSpec((_RB, 19), lambda i: (i, 0)),
            pl.BlockSpec((_RB, 2), lambda i: (i, 0)),
            pl.BlockSpec((19, 100), lambda i: (0, 0)),
            pl.BlockSpec((1, 100), lambda i: (0, 0)),
            pl.BlockSpec((100, 100), lambda i: (0, 0)),
        ],
        out_specs=[pl.BlockSpec((_RB, 128), lambda i: (i, 0)),
                   pl.BlockSpec((_RB, 1), lambda i: (i, 0))],
        out_shape=[jax.ShapeDtypeStruct((_N, 128), jnp.float32),
                   jax.ShapeDtypeStruct((_N, 1), jnp.float32)],
    )(x, degpair, W1, b1, Wc1)


def _tc_b_body(agg_ref, g1_ref, dis_ref, b1c_ref, Wc2_ref, o_ref):
    agg = agg_ref[...][:, :100]
    g1 = g1_ref[...][:, :100]
    dis = dis_ref[...]
    u1 = jnp.maximum(dis * (agg + g1) + b1c_ref[...], 0.0)
    t2 = jnp.dot(u1, Wc2_ref[...], preferred_element_type=jnp.float32)
    g2 = dis * t2
    z12 = jnp.zeros((_RB, 12), jnp.float32)
    z64 = jnp.zeros((_RB, 64), jnp.float32)
    o_ref[...] = jnp.concatenate(
        [g2[:, :20], z12, g2[:, 20:40], z12, z64], axis=1)


def _tc_b(aggfull, g1full, dis, b1c, Wc2):
    return pl.pallas_call(
        _tc_b_body,
        grid=(_N // _RB,),
        in_specs=[
            pl.BlockSpec((_RB, 128), lambda i: (i, 0)),
            pl.BlockSpec((_RB, 128), lambda i: (i, 0)),
            pl.BlockSpec((_RB, 1), lambda i: (i, 0)),
            pl.BlockSpec((1, 100), lambda i: (0, 0)),
            pl.BlockSpec((100, 40), lambda i: (0, 0)),
        ],
        out_specs=pl.BlockSpec((_RB, 128), lambda i: (i, 0)),
        out_shape=jax.ShapeDtypeStruct((_N, 128), jnp.float32),
    )(aggfull, g1full, dis, b1c, Wc2)


def _tc_c_body(agg_ref, g2_ref, dis_ref, b2c_ref, W2p_ref, b2p_ref, h3_ref):
    aggf = agg_ref[...]
    g2f = g2_ref[...]
    agg = jnp.concatenate([aggf[:, :20], aggf[:, 32:52]], axis=1)
    g2 = jnp.concatenate([g2f[:, :20], g2f[:, 32:52]], axis=1)
    dis = dis_ref[...]
    u2 = jnp.maximum(dis * (agg + g2) + b2c_ref[...], 0.0)
    h3_ref[...] = (
        jnp.dot(u2, W2p_ref[...], preferred_element_type=jnp.float32)
        + b2p_ref[...])


def _tc_c(aggfull, g2full, dis, b2c, W2p, b2p):
    return pl.pallas_call(
        _tc_c_body,
        grid=(_N // _RB,),
        in_specs=[
            pl.BlockSpec((_RB, 128), lambda i: (i, 0)),
            pl.BlockSpec((_RB, 128), lambda i: (i, 0)),
            pl.BlockSpec((_RB, 1), lambda i: (i, 0)),
            pl.BlockSpec((1, 40), lambda i: (0, 0)),
            pl.BlockSpec((40, 128), lambda i: (0, 0)),
            pl.BlockSpec((1, 128), lambda i: (0, 0)),
        ],
        out_specs=pl.BlockSpec((_RB, 128), lambda i: (i, 0)),
        out_shape=jax.ShapeDtypeStruct((_N + 176, 128), jnp.float32),
    )(aggfull, g2full, dis, b2c, W2p, b2p)


def _tc_d_body(p_ref, out_ref):
    p = p_ref[...]
    m = jnp.max(p, axis=0, keepdims=True)
    e = jnp.exp(p - m)
    ssum = jnp.sum(e, axis=0, keepdims=True)
    out_ref[...] = (e / ssum)[:, :6]


def _tc_d(pgrid):
    return pl.pallas_call(
        _tc_d_body,
        grid=(1,),
        in_specs=[pl.BlockSpec((_G, 8), lambda i: (0, 0))],
        out_specs=pl.BlockSpec((_G, 6), lambda i: (0, 0)),
        out_shape=jax.ShapeDtypeStruct((_G, 6), jnp.float32),
    )(pgrid)


def kernel(x, edge_index, batch, lin1_W, lin1_b, conv1_W, conv1_b,
           conv2_W, conv2_b, lin2_W, lin2_b):
    cols2 = edge_index[1].reshape(_NBLK, 128)
    ones128 = jnp.ones((128,), jnp.float32)
    z1 = jnp.zeros((_TPS,), jnp.float32)
    z2 = jnp.zeros((_TPS, _CW), jnp.float32)

    rows = edge_index[0]
    rows4 = [(rows * 4 + c).reshape(_NBLK, 128) for c in range(4)]

    deg_flat = _deg_kernel(cols2, ones128, z1)
    degpair = deg_flat.reshape(2, _NP).transpose(1, 0)

    g1full, dis = _tc_a(x, degpair, lin1_W, lin1_b[None, :], conv1_W)
    agg1 = _agg4(*rows4, cols2, z2, g1full.reshape(4 * _N, 32))
    g2full = _tc_b(agg1, g1full, dis, conv1_b[None, :], conv2_W)
    agg2 = _agg2(rows4[0], rows4[1], cols2, z2, g2full.reshape(4 * _N, 32))

    W2p = jnp.concatenate([lin2_W, jnp.zeros((40, 122), jnp.float32)], axis=1)
    b2p = jnp.concatenate(
        [lin2_b, jnp.ones((122,), jnp.float32)])[None, :]
    h3full = _tc_c(agg2, g2full, dis, conv2_b[None, :], W2p, b2p)

    pf = _seg_kernel(h3full.reshape(-1), batch)
    return _tc_d(pf.reshape(_G, 8))


# speedup vs baseline: 1.1456x; 1.1456x over previous
import functools

import jax
import jax.numpy as jnp
from jax import lax
from jax.experimental import pallas as pl
from jax.experimental.pallas import tpu as pltpu
from jax.experimental.pallas import tpu_sc as plsc

_N = 50000
_E = 800000
_G = 4096
_NP = 51200
_TPS = _NP // 16
_CW = 32
_NBLK = _E // 128
_NMAC = _NBLK // 5
_NPAIR = _NMAC // 2
_RB = 1000

_mesh = plsc.VectorSubcoreMesh(core_axis_name="c", subcore_axis_name="s")


@functools.partial(
    pl.kernel,
    out_type=jax.ShapeDtypeStruct((2 * _NP,), jnp.float32),
    mesh=_mesh,
    compiler_params=pltpu.CompilerParams(use_tc_tiling_on_sc=False),
    scratch_types=[
        pltpu.VMEM((5, 128), jnp.int32),
        pltpu.VMEM((5, 128), jnp.int32),
        pltpu.VMEM((128,), jnp.float32),
        pltpu.VMEM_SHARED((_NP,), jnp.float32),
        pltpu.SemaphoreType.DMA,
        pltpu.SemaphoreType.DMA,
    ],
)
def _deg_kernel(cols2_hbm, ones_hbm, z1_hbm, deg_out,
                ci0, ci1, ones_v, acc, isem, ssem):
    c = lax.axis_index("c")
    s = lax.axis_index("s")
    w = c * 16 + s
    pltpu.sync_copy(z1_hbm, acc.at[pl.ds(s * _TPS, _TPS)])
    pltpu.sync_copy(ones_hbm, ones_v)
    plsc.subcore_barrier()
    plo = (w * _NPAIR) // 32
    phi = ((w + 1) * _NPAIR) // 32

    def scat5(ci):
        sd = [pltpu.async_copy(ones_v, acc.at[ci.at[j]], ssem, add=True)
              for j in range(5)]
        for d in sd:
            d.wait()

    pltpu.sync_copy(cols2_hbm.at[pl.ds(2 * plo * 5, 5)], ci0)

    def pair(mp, _):
        m1 = 2 * mp + 1
        d1 = pltpu.async_copy(cols2_hbm.at[pl.ds(m1 * 5, 5)], ci1, isem)
        scat5(ci0)
        d1.wait()
        nxt = jnp.minimum(2 * mp + 2, _NMAC - 1)
        d0 = pltpu.async_copy(cols2_hbm.at[pl.ds(nxt * 5, 5)], ci0, isem)
        scat5(ci1)
        d0.wait()
        return 0

    lax.fori_loop(plo, phi, pair, 0)
    plsc.subcore_barrier()
    pltpu.sync_copy(acc.at[pl.ds(s * _TPS, _TPS)],
                    deg_out.at[pl.ds(c * _NP + s * _TPS, _TPS)])


def _make_agg(nchunks):
    per_core = nchunks // 2

    @functools.partial(
        pl.kernel,
        out_type=jax.ShapeDtypeStruct((_NP, 128), jnp.float32),
        mesh=_mesh,
        compiler_params=pltpu.CompilerParams(use_tc_tiling_on_sc=False),
        scratch_types=[
            pltpu.VMEM((5, 128), jnp.int32),
            pltpu.VMEM((5, 128), jnp.int32),
            pltpu.VMEM((5, 128), jnp.int32),
            pltpu.VMEM((5, 128), jnp.int32),
            pltpu.VMEM((5, 128, _CW), jnp.float32),
            pltpu.VMEM_SHARED((_NP, _CW), jnp.float32),
            pltpu.SemaphoreType.DMA,
            pltpu.SemaphoreType.DMA,
            pltpu.SemaphoreType.DMA,
        ],
    )
    def _agg(*args):
        rows4 = args[:nchunks]
        (cols2_hbm, z2_hbm, gflat_hbm, aggfull,
         ri0, ci0, ri1, ci1, buf, acc, isem, gsem, ssem) = args[nchunks:]
        c = lax.axis_index("c")
        s = lax.axis_index("s")
        plo = (s * _NPAIR) // 16
        phi = ((s + 1) * _NPAIR) // 16

        def one_pass(chunk):
            col0 = _CW * chunk
            r4 = rows4[chunk]
            pltpu.sync_copy(z2_hbm, acc.at[pl.ds(s * _TPS, _TPS)])
            plsc.subcore_barrier()

            def process(ri, ci):
                gd = [pltpu.async_copy(gflat_hbm.at[ri.at[j]], buf.at[j],
                                       gsem) for j in range(5)]
                sd = []
                for j in range(5):
                    gd[j].wait()
                    sd.append(pltpu.async_copy(buf.at[j], acc.at[ci.at[j]],
                                               ssem, add=True))
                for d in sd:
                    d.wait()

            pltpu.sync_copy(r4.at[pl.ds(2 * plo * 5, 5)], ri0)
            pltpu.sync_copy(cols2_hbm.at[pl.ds(2 * plo * 5, 5)], ci0)

            def pair(mp, _):
                m1 = 2 * mp + 1
                d1r = pltpu.async_copy(r4.at[pl.ds(m1 * 5, 5)], ri1, isem)
                d1c = pltpu.async_copy(cols2_hbm.at[pl.ds(m1 * 5, 5)], ci1,
                                       isem)
                process(ri0, ci0)
                d1r.wait()
                d1c.wait()
                nxt = jnp.minimum(2 * mp + 2, _NMAC - 1)
                d0r = pltpu.async_copy(r4.at[pl.ds(nxt * 5, 5)], ri0, isem)
                d0c = pltpu.async_copy(cols2_hbm.at[pl.ds(nxt * 5, 5)], ci0,
                                       isem)
                process(ri1, ci1)
                d0r.wait()
                d0c.wait()
                return 0

            lax.fori_loop(plo, phi, pair, 0)
            plsc.subcore_barrier()
            pltpu.sync_copy(
                acc.at[pl.ds(s * _TPS, _TPS)],
                aggfull.at[pl.ds(s * _TPS, _TPS), pl.ds(col0, _CW)])

        for core_id in range(2):
            @pl.when(c == core_id)
            def _(core_id=core_id):
                for k in range(per_core):
                    one_pass(core_id * per_core + k)

    return _agg


_agg4 = _make_agg(4)
_agg2 = _make_agg(2)


@functools.partial(
    pl.kernel,
    out_type=jax.ShapeDtypeStruct((_G * 8,), jnp.float32),
    mesh=_mesh,
    compiler_params=pltpu.CompilerParams(use_tc_tiling_on_sc=False,
                                         needs_layout_passes=False),
    scratch_types=[
        pltpu.VMEM((_N + 16,), jnp.int32),
        pltpu.VMEM((16656,), jnp.float32),
        pltpu.VMEM((1040,), jnp.float32),
    ],
)
def _seg_kernel(h3f_hbm, batch_hbm, pf_out, batch_v, buf_v, outl_v):
    w = lax.axis_index("s") * 2 + lax.axis_index("c")
    g0 = w * 128
    pltpu.sync_copy(batch_hbm, batch_v.at[pl.ds(0, _N)])

    def _bat(i):
        return batch_v[pl.ds(i, 16)][0]

    def _lower_bound(target):
        def body(_, st):
            lo, hi = st
            mid = (lo + hi) // 2
            big = _bat(mid) >= target
            nlo = jnp.where(big, lo, mid + 1)
            nhi = jnp.where(big, mid, hi)
            keep = lo < hi
            return (jnp.where(keep, nlo, lo), jnp.where(keep, nhi, hi))

        return lax.fori_loop(0, 17, body, (jnp.int32(0), jnp.int32(_N)))[0]

    lo = _lower_bound(g0)
    hi = _lower_bound(g0 + 128)

    ones16 = jnp.ones((16,), jnp.float32)

    def initb(k, _):
        outl_v[pl.ds(k * 16, 16)] = ones16
        return 0

    lax.fori_loop(0, 65, initb, 0)

    lane = lax.iota(jnp.int32, 16)
    lmask = lane < 8
    nblk = (hi - lo + 127) // 128

    def outer(t, _):
        i0 = lo + t * 128
        pltpu.sync_copy(h3f_hbm.at[pl.ds(i0 * 128, 16640)], buf_v.at[pl.ds(0, 16640)])
        nn = jnp.minimum(hi - i0, 128)

        def inner(j, _2):
            b = _bat(i0 + j)
            idxv = (b - g0) * 8 + lane
            old = plsc.load_gather(outl_v, [idxv], mask=lmask)
            v = buf_v[pl.ds(j * 128, 16)]
            v = jnp.where(lmask, v, 1.0)
            old = jnp.where(lmask, old, 1.0)
            plsc.store_scatter(outl_v, [idxv], old * v, mask=lmask)
            return 0

        lax.fori_loop(0, nn, inner, 0)
        return 0

    lax.fori_loop(0, nblk, outer, 0)
    pltpu.sync_copy(outl_v.at[pl.ds(0, 1024)], pf_out.at[pl.ds(w * 1024, 1024)])


def _tc_a_body(x_ref, degp_ref, W1_ref, b1_ref, Wc1_ref, gf_ref, dis_ref):
    degp = degp_ref[...]
    deg = degp[:, 0] + degp[:, 1] + 1.0
    dis = lax.rsqrt(deg)[:, None]
    h = jnp.maximum(
        jnp.dot(x_ref[...], W1_ref[...], preferred_element_type=jnp.float32)
        + b1_ref[...], 0.0)
    t1 = jnp.dot(h, Wc1_ref[...], preferred_element_type=jnp.float32)
    g = dis * t1
    gf_ref[...] = jnp.concatenate(
        [g, jnp.zeros((_RB, 28), jnp.float32)], axis=1)
    dis_ref[...] = dis


def _tc_a(x, degpair, W1, b1, Wc1):
    return pl.pallas_call(
        _tc_a_body,
        grid=(_N // _RB,),
        in_specs=[
            pl.BlockSpec((_RB, 19), lambda i: (i, 0)),
            pl.BlockSpec((_RB, 2), lambda i: (i, 0)),
            pl.BlockSpec((19, 100), lambda i: (0, 0)),
            pl.BlockSpec((1, 100), lambda i: (0, 0)),
            pl.BlockSpec((100, 100), lambda i: (0, 0)),
        ],
        out_specs=[pl.BlockSpec((_RB, 128), lambda i: (i, 0)),
                   pl.BlockSpec((_RB, 1), lambda i: (i, 0))],
        out_shape=[jax.ShapeDtypeStruct((_N, 128), jnp.float32),
                   jax.ShapeDtypeStruct((_N, 1), jnp.float32)],
    )(x, degpair, W1, b1, Wc1)


def _tc_b_body(agg_ref, g1_ref, dis_ref, b1c_ref, Wc2_ref, o_ref):
    agg = agg_ref[...][:, :100]
    g1 = g1_ref[...][:, :100]
    dis = dis_ref[...]
    u1 = jnp.maximum(dis * (agg + g1) + b1c_ref[...], 0.0)
    t2 = jnp.dot(u1, Wc2_ref[...], preferred_element_type=jnp.float32)
    g2 = dis * t2
    z12 = jnp.zeros((_RB, 12), jnp.float32)
    z64 = jnp.zeros((_RB, 64), jnp.float32)
    o_ref[...] = jnp.concatenate(
        [g2[:, :20], z12, g2[:, 20:40], z12, z64], axis=1)


def _tc_b(aggfull, g1full, dis, b1c, Wc2):
    return pl.pallas_call(
        _tc_b_body,
        grid=(_N // _RB,),
        in_specs=[
            pl.BlockSpec((_RB, 128), lambda i: (i, 0)),
            pl.BlockSpec((_RB, 128), lambda i: (i, 0)),
            pl.BlockSpec((_RB, 1), lambda i: (i, 0)),
            pl.BlockSpec((1, 100), lambda i: (0, 0)),
            pl.BlockSpec((100, 40), lambda i: (0, 0)),
        ],
        out_specs=pl.BlockSpec((_RB, 128), lambda i: (i, 0)),
        out_shape=jax.ShapeDtypeStruct((_N, 128), jnp.float32),
    )(aggfull, g1full, dis, b1c, Wc2)


def _tc_c_body(agg_ref, g2_ref, dis_ref, b2c_ref, W2p_ref, b2p_ref, h3_ref):
    aggf = agg_ref[...]
    g2f = g2_ref[...]
    agg = jnp.concatenate([aggf[:, :20], aggf[:, 32:52]], axis=1)
    g2 = jnp.concatenate([g2f[:, :20], g2f[:, 32:52]], axis=1)
    dis = dis_ref[...]
    u2 = jnp.maximum(dis * (agg + g2) + b2c_ref[...], 0.0)
    h3_ref[...] = (
        jnp.dot(u2, W2p_ref[...], preferred_element_type=jnp.float32)
        + b2p_ref[...])


def _tc_c(aggfull, g2full, dis, b2c, W2p, b2p):
    return pl.pallas_call(
        _tc_c_body,
        grid=(_N // _RB,),
        in_specs=[
            pl.BlockSpec((_RB, 128), lambda i: (i, 0)),
            pl.BlockSpec((_RB, 128), lambda i: (i, 0)),
            pl.BlockSpec((_RB, 1), lambda i: (i, 0)),
            pl.BlockSpec((1, 40), lambda i: (0, 0)),
            pl.BlockSpec((40, 128), lambda i: (0, 0)),
            pl.BlockSpec((1, 128), lambda i: (0, 0)),
        ],
        out_specs=pl.BlockSpec((_RB, 128), lambda i: (i, 0)),
        out_shape=jax.ShapeDtypeStruct((_N + 176, 128), jnp.float32),
    )(aggfull, g2full, dis, b2c, W2p, b2p)


def _tc_d_body(p_ref, out_ref):
    p = p_ref[...]
    m = jnp.max(p, axis=0, keepdims=True)
    e = jnp.exp(p - m)
    ssum = jnp.sum(e, axis=0, keepdims=True)
    out_ref[...] = (e / ssum)[:, :6]


def _tc_d(pgrid):
    return pl.pallas_call(
        _tc_d_body,
        grid=(1,),
        in_specs=[pl.BlockSpec((_G, 8), lambda i: (0, 0))],
        out_specs=pl.BlockSpec((_G, 6), lambda i: (0, 0)),
        out_shape=jax.ShapeDtypeStruct((_G, 6), jnp.float32),
    )(pgrid)


def kernel(x, edge_index, batch, lin1_W, lin1_b, conv1_W, conv1_b,
           conv2_W, conv2_b, lin2_W, lin2_b):
    cols2 = edge_index[1].reshape(_NBLK, 128)
    ones128 = jnp.ones((128,), jnp.float32)
    z1 = jnp.zeros((_TPS,), jnp.float32)
    z2 = jnp.zeros((_TPS, _CW), jnp.float32)

    rows = edge_index[0]
    rows4 = [(rows * 4 + c).reshape(_NBLK, 128) for c in range(4)]

    deg_flat = _deg_kernel(cols2, ones128, z1)
    degpair = deg_flat.reshape(2, _NP).transpose(1, 0)

    g1full, dis = _tc_a(x, degpair, lin1_W, lin1_b[None, :], conv1_W)
    agg1 = _agg4(*rows4, cols2, z2, g1full.reshape(4 * _N, 32))
    g2full = _tc_b(agg1, g1full, dis, conv1_b[None, :], conv2_W)
    agg2 = _agg2(rows4[0], rows4[1], cols2, z2, g2full.reshape(4 * _N, 32))

    W2p = jnp.concatenate([lin2_W, jnp.zeros((40, 122), jnp.float32)], axis=1)
    b2p = jnp.concatenate(
        [lin2_b, jnp.ones((122,), jnp.float32)])[None, :]
    h3full = _tc_c(agg2, g2full, dis, conv2_b[None, :], W2p, b2p)

    pf = _seg_kernel(h3full.reshape(-1), batch)
    return _tc_d(pf.reshape(_G, 8))


# speedup vs baseline: 1.2180x; 1.0632x over previous
import functools

import jax
import jax.numpy as jnp
from jax import lax
from jax.experimental import pallas as pl
from jax.experimental.pallas import tpu as pltpu
from jax.experimental.pallas import tpu_sc as plsc

_N = 50000
_E = 800000
_G = 4096
_NP = 51200
_TPS = _NP // 16
_CW = 32
_NBLK = _E // 128
_NMAC = _NBLK // 5
_NPAIR = _NMAC // 2
_RB = 2000

_mesh = plsc.VectorSubcoreMesh(core_axis_name="c", subcore_axis_name="s")


@functools.partial(
    pl.kernel,
    out_type=jax.ShapeDtypeStruct((2 * _NP,), jnp.float32),
    mesh=_mesh,
    compiler_params=pltpu.CompilerParams(use_tc_tiling_on_sc=False),
    scratch_types=[
        pltpu.VMEM((5, 128), jnp.int32),
        pltpu.VMEM((5, 128), jnp.int32),
        pltpu.VMEM((128,), jnp.float32),
        pltpu.VMEM_SHARED((_NP,), jnp.float32),
        pltpu.SemaphoreType.DMA,
        pltpu.SemaphoreType.DMA,
    ],
)
def _deg_kernel(cols2_hbm, ones_hbm, z1_hbm, deg_out,
                ci0, ci1, ones_v, acc, isem, ssem):
    c = lax.axis_index("c")
    s = lax.axis_index("s")
    w = c * 16 + s
    pltpu.sync_copy(z1_hbm, acc.at[pl.ds(s * _TPS, _TPS)])
    pltpu.sync_copy(ones_hbm, ones_v)
    plsc.subcore_barrier()
    plo = (w * _NPAIR) // 32
    phi = ((w + 1) * _NPAIR) // 32

    def scat5(ci):
        sd = [pltpu.async_copy(ones_v, acc.at[ci.at[j]], ssem, add=True)
              for j in range(5)]
        for d in sd:
            d.wait()

    pltpu.sync_copy(cols2_hbm.at[pl.ds(2 * plo * 5, 5)], ci0)

    def pair(mp, _):
        m1 = 2 * mp + 1
        d1 = pltpu.async_copy(cols2_hbm.at[pl.ds(m1 * 5, 5)], ci1, isem)
        scat5(ci0)
        d1.wait()
        nxt = jnp.minimum(2 * mp + 2, _NMAC - 1)
        d0 = pltpu.async_copy(cols2_hbm.at[pl.ds(nxt * 5, 5)], ci0, isem)
        scat5(ci1)
        d0.wait()
        return 0

    lax.fori_loop(plo, phi, pair, 0)
    plsc.subcore_barrier()
    pltpu.sync_copy(acc.at[pl.ds(s * _TPS, _TPS)],
                    deg_out.at[pl.ds(c * _NP + s * _TPS, _TPS)])


def _make_agg(nchunks):
    per_core = nchunks // 2

    @functools.partial(
        pl.kernel,
        out_type=jax.ShapeDtypeStruct((_NP, 128), jnp.float32),
        mesh=_mesh,
        compiler_params=pltpu.CompilerParams(use_tc_tiling_on_sc=False),
        scratch_types=[
            pltpu.VMEM((5, 128), jnp.int32),
            pltpu.VMEM((5, 128), jnp.int32),
            pltpu.VMEM((5, 128), jnp.int32),
            pltpu.VMEM((5, 128), jnp.int32),
            pltpu.VMEM((5, 128, _CW), jnp.float32),
            pltpu.VMEM_SHARED((_NP, _CW), jnp.float32),
            pltpu.SemaphoreType.DMA,
            pltpu.SemaphoreType.DMA,
            pltpu.SemaphoreType.DMA,
        ],
    )
    def _agg(*args):
        rows4 = args[:nchunks]
        (cols2_hbm, z2_hbm, gflat_hbm, aggfull,
         ri0, ci0, ri1, ci1, buf, acc, isem, gsem, ssem) = args[nchunks:]
        c = lax.axis_index("c")
        s = lax.axis_index("s")
        plo = (s * _NPAIR) // 16
        phi = ((s + 1) * _NPAIR) // 16

        def one_pass(chunk):
            col0 = _CW * chunk
            r4 = rows4[chunk]
            pltpu.sync_copy(z2_hbm, acc.at[pl.ds(s * _TPS, _TPS)])
            plsc.subcore_barrier()

            def process(ri, ci):
                gd = [pltpu.async_copy(gflat_hbm.at[ri.at[j]], buf.at[j],
                                       gsem) for j in range(5)]
                sd = []
                for j in range(5):
                    gd[j].wait()
                    sd.append(pltpu.async_copy(buf.at[j], acc.at[ci.at[j]],
                                               ssem, add=True))
                for d in sd:
                    d.wait()

            pltpu.sync_copy(r4.at[pl.ds(2 * plo * 5, 5)], ri0)
            pltpu.sync_copy(cols2_hbm.at[pl.ds(2 * plo * 5, 5)], ci0)

            def pair(mp, _):
                m1 = 2 * mp + 1
                d1r = pltpu.async_copy(r4.at[pl.ds(m1 * 5, 5)], ri1, isem)
                d1c = pltpu.async_copy(cols2_hbm.at[pl.ds(m1 * 5, 5)], ci1,
                                       isem)
                process(ri0, ci0)
                d1r.wait()
                d1c.wait()
                nxt = jnp.minimum(2 * mp + 2, _NMAC - 1)
                d0r = pltpu.async_copy(r4.at[pl.ds(nxt * 5, 5)], ri0, isem)
                d0c = pltpu.async_copy(cols2_hbm.at[pl.ds(nxt * 5, 5)], ci0,
                                       isem)
                process(ri1, ci1)
                d0r.wait()
                d0c.wait()
                return 0

            lax.fori_loop(plo, phi, pair, 0)
            plsc.subcore_barrier()
            pltpu.sync_copy(
                acc.at[pl.ds(s * _TPS, _TPS)],
                aggfull.at[pl.ds(s * _TPS, _TPS), pl.ds(col0, _CW)])

        for core_id in range(2):
            @pl.when(c == core_id)
            def _(core_id=core_id):
                for k in range(per_core):
                    one_pass(core_id * per_core + k)

    return _agg


_agg4 = _make_agg(4)
_agg2 = _make_agg(2)


@functools.partial(
    pl.kernel,
    out_type=jax.ShapeDtypeStruct((_G * 8,), jnp.float32),
    mesh=_mesh,
    compiler_params=pltpu.CompilerParams(use_tc_tiling_on_sc=False,
                                         needs_layout_passes=False),
    scratch_types=[
        pltpu.VMEM((_N + 16,), jnp.int32),
        pltpu.VMEM((16656,), jnp.float32),
        pltpu.VMEM((1040,), jnp.float32),
    ],
)
def _seg_kernel(h3f_hbm, batch_hbm, pf_out, batch_v, buf_v, outl_v):
    w = lax.axis_index("s") * 2 + lax.axis_index("c")
    g0 = w * 128
    pltpu.sync_copy(batch_hbm, batch_v.at[pl.ds(0, _N)])

    def _bat(i):
        return batch_v[pl.ds(i, 16)][0]

    def _lower_bound(target):
        def body(_, st):
            lo, hi = st
            mid = (lo + hi) // 2
            big = _bat(mid) >= target
            nlo = jnp.where(big, lo, mid + 1)
            nhi = jnp.where(big, mid, hi)
            keep = lo < hi
            return (jnp.where(keep, nlo, lo), jnp.where(keep, nhi, hi))

        return lax.fori_loop(0, 17, body, (jnp.int32(0), jnp.int32(_N)))[0]

    lo = _lower_bound(g0)
    hi = _lower_bound(g0 + 128)

    ones16 = jnp.ones((16,), jnp.float32)

    def initb(k, _):
        outl_v[pl.ds(k * 16, 16)] = ones16
        return 0

    lax.fori_loop(0, 65, initb, 0)

    lane = lax.iota(jnp.int32, 16)
    lmask = lane < 8
    nblk = (hi - lo + 127) // 128

    def outer(t, _):
        i0 = lo + t * 128
        pltpu.sync_copy(h3f_hbm.at[pl.ds(i0 * 128, 16640)], buf_v.at[pl.ds(0, 16640)])
        nn = jnp.minimum(hi - i0, 128)

        def inner(j, _2):
            b = _bat(i0 + j)
            idxv = (b - g0) * 8 + lane
            old = plsc.load_gather(outl_v, [idxv], mask=lmask)
            v = buf_v[pl.ds(j * 128, 16)]
            v = jnp.where(lmask, v, 1.0)
            old = jnp.where(lmask, old, 1.0)
            plsc.store_scatter(outl_v, [idxv], old * v, mask=lmask)
            return 0

        lax.fori_loop(0, nn, inner, 0)
        return 0

    lax.fori_loop(0, nblk, outer, 0)
    pltpu.sync_copy(outl_v.at[pl.ds(0, 1024)], pf_out.at[pl.ds(w * 1024, 1024)])


def _tc_a_body(x_ref, degp_ref, W1_ref, b1_ref, Wc1_ref, gf_ref, dis_ref):
    degp = degp_ref[...]
    deg = degp[:, 0] + degp[:, 1] + 1.0
    dis = lax.rsqrt(deg)[:, None]
    h = jnp.maximum(
        jnp.dot(x_ref[...], W1_ref[...], preferred_element_type=jnp.float32)
        + b1_ref[...], 0.0)
    t1 = jnp.dot(h, Wc1_ref[...], preferred_element_type=jnp.float32)
    g = dis * t1
    gf_ref[...] = jnp.concatenate(
        [g, jnp.zeros((_RB, 28), jnp.float32)], axis=1)
    dis_ref[...] = dis


def _tc_a(x, degpair, W1, b1, Wc1):
    return pl.pallas_call(
        _tc_a_body,
        grid=(_N // _RB,),
        in_specs=[
            pl.BlockSpec((_RB, 19), lambda i: (i, 0)),
            pl.BlockSpec((_RB, 2), lambda i: (i, 0)),
            pl.BlockSpec((19, 100), lambda i: (0, 0)),
            pl.BlockSpec((1, 100), lambda i: (0, 0)),
            pl.BlockSpec((100, 100), lambda i: (0, 0)),
        ],
        out_specs=[pl.BlockSpec((_RB, 128), lambda i: (i, 0)),
                   pl.BlockSpec((_RB, 1), lambda i: (i, 0))],
        out_shape=[jax.ShapeDtypeStruct((_N, 128), jnp.float32),
                   jax.ShapeDtypeStruct((_N, 1), jnp.float32)],
    )(x, degpair, W1, b1, Wc1)


def _tc_b_body(agg_ref, g1_ref, dis_ref, b1c_ref, Wc2_ref, o_ref):
    agg = agg_ref[...][:, :100]
    g1 = g1_ref[...][:, :100]
    dis = dis_ref[...]
    u1 = jnp.maximum(dis * (agg + g1) + b1c_ref[...], 0.0)
    t2 = jnp.dot(u1, Wc2_ref[...], preferred_element_type=jnp.float32)
    g2 = dis * t2
    z12 = jnp.zeros((_RB, 12), jnp.float32)
    z64 = jnp.zeros((_RB, 64), jnp.float32)
    o_ref[...] = jnp.concatenate(
        [g2[:, :20], z12, g2[:, 20:40], z12, z64], axis=1)


def _tc_b(aggfull, g1full, dis, b1c, Wc2):
    return pl.pallas_call(
        _tc_b_body,
        grid=(_N // _RB,),
        in_specs=[
            pl.BlockSpec((_RB, 128), lambda i: (i, 0)),
            pl.BlockSpec((_RB, 128), lambda i: (i, 0)),
            pl.BlockSpec((_RB, 1), lambda i: (i, 0)),
            pl.BlockSpec((1, 100), lambda i: (0, 0)),
            pl.BlockSpec((100, 40), lambda i: (0, 0)),
        ],
        out_specs=pl.BlockSpec((_RB, 128), lambda i: (i, 0)),
        out_shape=jax.ShapeDtypeStruct((_N, 128), jnp.float32),
    )(aggfull, g1full, dis, b1c, Wc2)


def _tc_c_body(agg_ref, g2_ref, dis_ref, b2c_ref, W2p_ref, b2p_ref, h3_ref):
    aggf = agg_ref[...]
    g2f = g2_ref[...]
    agg = jnp.concatenate([aggf[:, :20], aggf[:, 32:52]], axis=1)
    g2 = jnp.concatenate([g2f[:, :20], g2f[:, 32:52]], axis=1)
    dis = dis_ref[...]
    u2 = jnp.maximum(dis * (agg + g2) + b2c_ref[...], 0.0)
    h3_ref[...] = (
        jnp.dot(u2, W2p_ref[...], preferred_element_type=jnp.float32)
        + b2p_ref[...])


def _tc_c(aggfull, g2full, dis, b2c, W2p, b2p):
    return pl.pallas_call(
        _tc_c_body,
        grid=(_N // _RB,),
        in_specs=[
            pl.BlockSpec((_RB, 128), lambda i: (i, 0)),
            pl.BlockSpec((_RB, 128), lambda i: (i, 0)),
            pl.BlockSpec((_RB, 1), lambda i: (i, 0)),
            pl.BlockSpec((1, 40), lambda i: (0, 0)),
            pl.BlockSpec((40, 128), lambda i: (0, 0)),
            pl.BlockSpec((1, 128), lambda i: (0, 0)),
        ],
        out_specs=pl.BlockSpec((_RB, 128), lambda i: (i, 0)),
        out_shape=jax.ShapeDtypeStruct((_N + 176, 128), jnp.float32),
    )(aggfull, g2full, dis, b2c, W2p, b2p)


def _tc_d_body(p_ref, out_ref):
    p = p_ref[...]
    m = jnp.max(p, axis=0, keepdims=True)
    e = jnp.exp(p - m)
    ssum = jnp.sum(e, axis=0, keepdims=True)
    out_ref[...] = (e / ssum)[:, :6]


def _tc_d(pgrid):
    return pl.pallas_call(
        _tc_d_body,
        grid=(1,),
        in_specs=[pl.BlockSpec((_G, 8), lambda i: (0, 0))],
        out_specs=pl.BlockSpec((_G, 6), lambda i: (0, 0)),
        out_shape=jax.ShapeDtypeStruct((_G, 6), jnp.float32),
    )(pgrid)


def kernel(x, edge_index, batch, lin1_W, lin1_b, conv1_W, conv1_b,
           conv2_W, conv2_b, lin2_W, lin2_b):
    cols2 = edge_index[1].reshape(_NBLK, 128)
    ones128 = jnp.ones((128,), jnp.float32)
    z1 = jnp.zeros((_TPS,), jnp.float32)
    z2 = jnp.zeros((_TPS, _CW), jnp.float32)

    rows = edge_index[0]
    rows4 = [(rows * 4 + c).reshape(_NBLK, 128) for c in range(4)]

    deg_flat = _deg_kernel(cols2, ones128, z1)
    degpair = deg_flat.reshape(2, _NP).transpose(1, 0)

    g1full, dis = _tc_a(x, degpair, lin1_W, lin1_b[None, :], conv1_W)
    agg1 = _agg4(*rows4, cols2, z2, g1full.reshape(4 * _N, 32))
    g2full = _tc_b(agg1, g1full, dis, conv1_b[None, :], conv2_W)
    agg2 = _agg2(rows4[0], rows4[1], cols2, z2, g2full.reshape(4 * _N, 32))

    W2p = jnp.concatenate([lin2_W, jnp.zeros((40, 122), jnp.float32)], axis=1)
    b2p = jnp.concatenate(
        [lin2_b, jnp.ones((122,), jnp.float32)])[None, :]
    h3full = _tc_c(agg2, g2full, dis, conv2_b[None, :], W2p, b2p)

    pf = _seg_kernel(h3full.reshape(-1), batch)
    return _tc_d(pf.reshape(_G, 8))


# speedup vs baseline: 1.2474x; 1.0242x over previous
import functools

import jax
import jax.numpy as jnp
from jax import lax
from jax.experimental import pallas as pl
from jax.experimental.pallas import tpu as pltpu
from jax.experimental.pallas import tpu_sc as plsc

_N = 50000
_E = 800000
_G = 4096
_NP = 51200
_TPS = _NP // 16
_CW = 32
_NBLK = _E // 128
_NMAC = _NBLK // 5
_NPAIR = _NMAC // 2
_RB = 5000

_mesh = plsc.VectorSubcoreMesh(core_axis_name="c", subcore_axis_name="s")


@functools.partial(
    pl.kernel,
    out_type=jax.ShapeDtypeStruct((2 * _NP,), jnp.float32),
    mesh=_mesh,
    compiler_params=pltpu.CompilerParams(use_tc_tiling_on_sc=False),
    scratch_types=[
        pltpu.VMEM((5, 128), jnp.int32),
        pltpu.VMEM((5, 128), jnp.int32),
        pltpu.VMEM((128,), jnp.float32),
        pltpu.VMEM_SHARED((_NP,), jnp.float32),
        pltpu.SemaphoreType.DMA,
        pltpu.SemaphoreType.DMA,
    ],
)
def _deg_kernel(cols2_hbm, ones_hbm, z1_hbm, deg_out,
                ci0, ci1, ones_v, acc, isem, ssem):
    c = lax.axis_index("c")
    s = lax.axis_index("s")
    w = c * 16 + s
    pltpu.sync_copy(z1_hbm, acc.at[pl.ds(s * _TPS, _TPS)])
    pltpu.sync_copy(ones_hbm, ones_v)
    plsc.subcore_barrier()
    plo = (w * _NPAIR) // 32
    phi = ((w + 1) * _NPAIR) // 32

    def scat5(ci):
        sd = [pltpu.async_copy(ones_v, acc.at[ci.at[j]], ssem, add=True)
              for j in range(5)]
        for d in sd:
            d.wait()

    pltpu.sync_copy(cols2_hbm.at[pl.ds(2 * plo * 5, 5)], ci0)

    def pair(mp, _):
        m1 = 2 * mp + 1
        d1 = pltpu.async_copy(cols2_hbm.at[pl.ds(m1 * 5, 5)], ci1, isem)
        scat5(ci0)
        d1.wait()
        nxt = jnp.minimum(2 * mp + 2, _NMAC - 1)
        d0 = pltpu.async_copy(cols2_hbm.at[pl.ds(nxt * 5, 5)], ci0, isem)
        scat5(ci1)
        d0.wait()
        return 0

    lax.fori_loop(plo, phi, pair, 0)
    plsc.subcore_barrier()
    pltpu.sync_copy(acc.at[pl.ds(s * _TPS, _TPS)],
                    deg_out.at[pl.ds(c * _NP + s * _TPS, _TPS)])


def _make_agg(nchunks):
    per_core = nchunks // 2

    @functools.partial(
        pl.kernel,
        out_type=jax.ShapeDtypeStruct((_NP, 128), jnp.float32),
        mesh=_mesh,
        compiler_params=pltpu.CompilerParams(use_tc_tiling_on_sc=False),
        scratch_types=[
            pltpu.VMEM((5, 128), jnp.int32),
            pltpu.VMEM((5, 128), jnp.int32),
            pltpu.VMEM((5, 128), jnp.int32),
            pltpu.VMEM((5, 128), jnp.int32),
            pltpu.VMEM((5, 128, _CW), jnp.float32),
            pltpu.VMEM_SHARED((_NP, _CW), jnp.float32),
            pltpu.SemaphoreType.DMA,
            pltpu.SemaphoreType.DMA,
            pltpu.SemaphoreType.DMA,
        ],
    )
    def _agg(*args):
        rows4 = args[:nchunks]
        (cols2_hbm, z2_hbm, gflat_hbm, aggfull,
         ri0, ci0, ri1, ci1, buf, acc, isem, gsem, ssem) = args[nchunks:]
        c = lax.axis_index("c")
        s = lax.axis_index("s")
        plo = (s * _NPAIR) // 16
        phi = ((s + 1) * _NPAIR) // 16

        def one_pass(chunk):
            col0 = _CW * chunk
            r4 = rows4[chunk]
            pltpu.sync_copy(z2_hbm, acc.at[pl.ds(s * _TPS, _TPS)])
            plsc.subcore_barrier()

            def process(ri, ci):
                gd = [pltpu.async_copy(gflat_hbm.at[ri.at[j]], buf.at[j],
                                       gsem) for j in range(5)]
                sd = []
                for j in range(5):
                    gd[j].wait()
                    sd.append(pltpu.async_copy(buf.at[j], acc.at[ci.at[j]],
                                               ssem, add=True))
                for d in sd:
                    d.wait()

            pltpu.sync_copy(r4.at[pl.ds(2 * plo * 5, 5)], ri0)
            pltpu.sync_copy(cols2_hbm.at[pl.ds(2 * plo * 5, 5)], ci0)

            def pair(mp, _):
                m1 = 2 * mp + 1
                d1r = pltpu.async_copy(r4.at[pl.ds(m1 * 5, 5)], ri1, isem)
                d1c = pltpu.async_copy(cols2_hbm.at[pl.ds(m1 * 5, 5)], ci1,
                                       isem)
                process(ri0, ci0)
                d1r.wait()
                d1c.wait()
                nxt = jnp.minimum(2 * mp + 2, _NMAC - 1)
                d0r = pltpu.async_copy(r4.at[pl.ds(nxt * 5, 5)], ri0, isem)
                d0c = pltpu.async_copy(cols2_hbm.at[pl.ds(nxt * 5, 5)], ci0,
                                       isem)
                process(ri1, ci1)
                d0r.wait()
                d0c.wait()
                return 0

            lax.fori_loop(plo, phi, pair, 0)
            plsc.subcore_barrier()
            pltpu.sync_copy(
                acc.at[pl.ds(s * _TPS, _TPS)],
                aggfull.at[pl.ds(s * _TPS, _TPS), pl.ds(col0, _CW)])

        for core_id in range(2):
            @pl.when(c == core_id)
            def _(core_id=core_id):
                for k in range(per_core):
                    one_pass(core_id * per_core + k)

    return _agg


_agg4 = _make_agg(4)
_agg2 = _make_agg(2)


@functools.partial(
    pl.kernel,
    out_type=jax.ShapeDtypeStruct((_G * 8,), jnp.float32),
    mesh=_mesh,
    compiler_params=pltpu.CompilerParams(use_tc_tiling_on_sc=False,
                                         needs_layout_passes=False),
    scratch_types=[
        pltpu.VMEM((_N + 16,), jnp.int32),
        pltpu.VMEM((16656,), jnp.float32),
        pltpu.VMEM((1040,), jnp.float32),
    ],
)
def _seg_kernel(h3f_hbm, batch_hbm, pf_out, batch_v, buf_v, outl_v):
    w = lax.axis_index("s") * 2 + lax.axis_index("c")
    g0 = w * 128
    pltpu.sync_copy(batch_hbm, batch_v.at[pl.ds(0, _N)])

    def _bat(i):
        return batch_v[pl.ds(i, 16)][0]

    def _lower_bound(target):
        def body(_, st):
            lo, hi = st
            mid = (lo + hi) // 2
            big = _bat(mid) >= target
            nlo = jnp.where(big, lo, mid + 1)
            nhi = jnp.where(big, mid, hi)
            keep = lo < hi
            return (jnp.where(keep, nlo, lo), jnp.where(keep, nhi, hi))

        return lax.fori_loop(0, 17, body, (jnp.int32(0), jnp.int32(_N)))[0]

    lo = _lower_bound(g0)
    hi = _lower_bound(g0 + 128)

    ones16 = jnp.ones((16,), jnp.float32)

    def initb(k, _):
        outl_v[pl.ds(k * 16, 16)] = ones16
        return 0

    lax.fori_loop(0, 65, initb, 0)

    lane = lax.iota(jnp.int32, 16)
    lmask = lane < 8
    nblk = (hi - lo + 127) // 128

    def outer(t, _):
        i0 = lo + t * 128
        pltpu.sync_copy(h3f_hbm.at[pl.ds(i0 * 128, 16640)], buf_v.at[pl.ds(0, 16640)])
        nn = jnp.minimum(hi - i0, 128)

        def inner(j, _2):
            b = _bat(i0 + j)
            idxv = (b - g0) * 8 + lane
            old = plsc.load_gather(outl_v, [idxv], mask=lmask)
            v = buf_v[pl.ds(j * 128, 16)]
            v = jnp.where(lmask, v, 1.0)
            old = jnp.where(lmask, old, 1.0)
            plsc.store_scatter(outl_v, [idxv], old * v, mask=lmask)
            return 0

        lax.fori_loop(0, nn, inner, 0)
        return 0

    lax.fori_loop(0, nblk, outer, 0)
    pltpu.sync_copy(outl_v.at[pl.ds(0, 1024)], pf_out.at[pl.ds(w * 1024, 1024)])


def _tc_a_body(x_ref, degp_ref, W1_ref, b1_ref, Wc1_ref, gf_ref, dis_ref):
    degp = degp_ref[...]
    deg = degp[:, 0] + degp[:, 1] + 1.0
    dis = lax.rsqrt(deg)[:, None]
    h = jnp.maximum(
        jnp.dot(x_ref[...], W1_ref[...], preferred_element_type=jnp.float32)
        + b1_ref[...], 0.0)
    t1 = jnp.dot(h, Wc1_ref[...], preferred_element_type=jnp.float32)
    g = dis * t1
    gf_ref[...] = jnp.concatenate(
        [g, jnp.zeros((_RB, 28), jnp.float32)], axis=1)
    dis_ref[...] = dis


def _tc_a(x, degpair, W1, b1, Wc1):
    return pl.pallas_call(
        _tc_a_body,
        grid=(_N // _RB,),
        in_specs=[
            pl.BlockSpec((_RB, 19), lambda i: (i, 0)),
            pl.BlockSpec((_RB, 2), lambda i: (i, 0)),
            pl.BlockSpec((19, 100), lambda i: (0, 0)),
            pl.BlockSpec((1, 100), lambda i: (0, 0)),
            pl.BlockSpec((100, 100), lambda i: (0, 0)),
        ],
        out_specs=[pl.BlockSpec((_RB, 128), lambda i: (i, 0)),
                   pl.BlockSpec((_RB, 1), lambda i: (i, 0))],
        out_shape=[jax.ShapeDtypeStruct((_N, 128), jnp.float32),
                   jax.ShapeDtypeStruct((_N, 1), jnp.float32)],
    )(x, degpair, W1, b1, Wc1)


def _tc_b_body(agg_ref, g1_ref, dis_ref, b1c_ref, Wc2_ref, o_ref):
    agg = agg_ref[...][:, :100]
    g1 = g1_ref[...][:, :100]
    dis = dis_ref[...]
    u1 = jnp.maximum(dis * (agg + g1) + b1c_ref[...], 0.0)
    t2 = jnp.dot(u1, Wc2_ref[...], preferred_element_type=jnp.float32)
    g2 = dis * t2
    z12 = jnp.zeros((_RB, 12), jnp.float32)
    z64 = jnp.zeros((_RB, 64), jnp.float32)
    o_ref[...] = jnp.concatenate(
        [g2[:, :20], z12, g2[:, 20:40], z12, z64], axis=1)


def _tc_b(aggfull, g1full, dis, b1c, Wc2):
    return pl.pallas_call(
        _tc_b_body,
        grid=(_N // _RB,),
        in_specs=[
            pl.BlockSpec((_RB, 128), lambda i: (i, 0)),
            pl.BlockSpec((_RB, 128), lambda i: (i, 0)),
            pl.BlockSpec((_RB, 1), lambda i: (i, 0)),
            pl.BlockSpec((1, 100), lambda i: (0, 0)),
            pl.BlockSpec((100, 40), lambda i: (0, 0)),
        ],
        out_specs=pl.BlockSpec((_RB, 128), lambda i: (i, 0)),
        out_shape=jax.ShapeDtypeStruct((_N, 128), jnp.float32),
    )(aggfull, g1full, dis, b1c, Wc2)


def _tc_c_body(agg_ref, g2_ref, dis_ref, b2c_ref, W2p_ref, b2p_ref, h3_ref):
    aggf = agg_ref[...]
    g2f = g2_ref[...]
    agg = jnp.concatenate([aggf[:, :20], aggf[:, 32:52]], axis=1)
    g2 = jnp.concatenate([g2f[:, :20], g2f[:, 32:52]], axis=1)
    dis = dis_ref[...]
    u2 = jnp.maximum(dis * (agg + g2) + b2c_ref[...], 0.0)
    h3_ref[...] = (
        jnp.dot(u2, W2p_ref[...], preferred_element_type=jnp.float32)
        + b2p_ref[...])


def _tc_c(aggfull, g2full, dis, b2c, W2p, b2p):
    return pl.pallas_call(
        _tc_c_body,
        grid=(_N // _RB,),
        in_specs=[
            pl.BlockSpec((_RB, 128), lambda i: (i, 0)),
            pl.BlockSpec((_RB, 128), lambda i: (i, 0)),
            pl.BlockSpec((_RB, 1), lambda i: (i, 0)),
            pl.BlockSpec((1, 40), lambda i: (0, 0)),
            pl.BlockSpec((40, 128), lambda i: (0, 0)),
            pl.BlockSpec((1, 128), lambda i: (0, 0)),
        ],
        out_specs=pl.BlockSpec((_RB, 128), lambda i: (i, 0)),
        out_shape=jax.ShapeDtypeStruct((_N + 176, 128), jnp.float32),
    )(aggfull, g2full, dis, b2c, W2p, b2p)


def _tc_d_body(p_ref, out_ref):
    p = p_ref[...]
    m = jnp.max(p, axis=0, keepdims=True)
    e = jnp.exp(p - m)
    ssum = jnp.sum(e, axis=0, keepdims=True)
    out_ref[...] = (e / ssum)[:, :6]


def _tc_d(pgrid):
    return pl.pallas_call(
        _tc_d_body,
        grid=(1,),
        in_specs=[pl.BlockSpec((_G, 8), lambda i: (0, 0))],
        out_specs=pl.BlockSpec((_G, 6), lambda i: (0, 0)),
        out_shape=jax.ShapeDtypeStruct((_G, 6), jnp.float32),
    )(pgrid)


def kernel(x, edge_index, batch, lin1_W, lin1_b, conv1_W, conv1_b,
           conv2_W, conv2_b, lin2_W, lin2_b):
    cols2 = edge_index[1].reshape(_NBLK, 128)
    ones128 = jnp.ones((128,), jnp.float32)
    z1 = jnp.zeros((_TPS,), jnp.float32)
    z2 = jnp.zeros((_TPS, _CW), jnp.float32)

    rows = edge_index[0]
    rows4 = [(rows * 4 + c).reshape(_NBLK, 128) for c in range(4)]

    deg_flat = _deg_kernel(cols2, ones128, z1)
    degpair = deg_flat.reshape(2, _NP).transpose(1, 0)

    g1full, dis = _tc_a(x, degpair, lin1_W, lin1_b[None, :], conv1_W)
    agg1 = _agg4(*rows4, cols2, z2, g1full.reshape(4 * _N, 32))
    g2full = _tc_b(agg1, g1full, dis, conv1_b[None, :], conv2_W)
    agg2 = _agg2(rows4[0], rows4[1], cols2, z2, g2full.reshape(4 * _N, 32))

    W2p = jnp.concatenate([lin2_W, jnp.zeros((40, 122), jnp.float32)], axis=1)
    b2p = jnp.concatenate(
        [lin2_b, jnp.ones((122,), jnp.float32)])[None, :]
    h3full = _tc_c(agg2, g2full, dis, conv2_b[None, :], W2p, b2p)

    pf = _seg_kernel(h3full.reshape(-1), batch)
    return _tc_d(pf.reshape(_G, 8))
